# baseline (speedup 1.0000x reference)
import functools

import jax
import jax.numpy as jnp
import numpy as np
from jax import lax
from jax.experimental import pallas as pl
from jax.experimental.pallas import tpu as pltpu
from jax.experimental.pallas import tpu_sc as plsc

N = 10000
E = 160000
D = 128
D_ADD = 16
D_ATTR = 4

NC, NS = 2, 16
NW = NC * NS
C = 128
NCHUNK = E // C
CPW = -(-NCHUNK // NW)
NPAD = 10240
NPSC = NPAD // NC
NPT = NPSC // NS

_mesh = lambda: plsc.VectorSubcoreMesh(
    core_axis_name="c", subcore_axis_name="s", num_cores=NC, num_subcores=NS)


def _silu(t):
    return t / (1.0 + jnp.exp(-t))


def _combine(t, attr, bias):
    acc = bias
    for b in range(D_ATTR):
        acc = acc + t[:, b * D:(b + 1) * D] * attr[:, b:b + 1]
    return acc


DP = D // 2


def _gather_body(NCHUNK, CPW, x_hbm, rcv_hbm, snd_hbm, xr_hbm, xs_hbm,
                 idxr_v, idxs_v, rowsr_v, rowss_v, isem, gsem, wsem):
    wid = lax.axis_index("s") * NC + lax.axis_index("c")

    def fetch_idx(k, buf):
        cid = wid + jnp.minimum(k, CPW - 1) * NW
        base = jnp.minimum(cid, NCHUNK - 1) * C
        pltpu.async_copy(rcv_hbm.at[pl.ds(base, C)], idxr_v.at[buf], isem.at[buf])
        pltpu.async_copy(snd_hbm.at[pl.ds(base, C)], idxs_v.at[buf], isem.at[buf])

    def wait_idx(buf):
        pltpu.make_async_copy(rcv_hbm.at[pl.ds(0, C)], idxr_v.at[buf],
                              isem.at[buf]).wait()
        pltpu.make_async_copy(snd_hbm.at[pl.ds(0, C)], idxs_v.at[buf],
                              isem.at[buf]).wait()

    def wait_wb(buf):
        pltpu.make_async_copy(rowsr_v.at[buf], xr_hbm.at[pl.ds(0, C)],
                              wsem.at[buf]).wait()
        pltpu.make_async_copy(rowss_v.at[buf], xs_hbm.at[pl.ds(0, C)],
                              wsem.at[buf]).wait()

    fetch_idx(0, 0)

    def chunk(k, _):
        buf = k & 1
        cid = wid + k * NW

        @pl.when(k >= 2)
        def _():
            wait_wb(buf)

        wait_idx(buf)
        cr = pltpu.async_copy(x_hbm.at[idxr_v.at[buf]], rowsr_v.at[buf],
                              gsem.at[buf])
        cs = pltpu.async_copy(x_hbm.at[idxs_v.at[buf]], rowss_v.at[buf],
                              gsem.at[buf])

        @pl.when(k + 1 < CPW)
        def _():
            fetch_idx(k + 1, 1 - buf)

        cr.wait()
        cs.wait()

        @pl.when(cid < NCHUNK)
        def _():
            base = cid * C
            pltpu.async_copy(rowsr_v.at[buf], xr_hbm.at[pl.ds(base, C)],
                             wsem.at[buf])
            pltpu.async_copy(rowss_v.at[buf], xs_hbm.at[pl.ds(base, C)],
                             wsem.at[buf])
        return 0

    lax.fori_loop(0, CPW, chunk, 0)
    wait_wb((CPW - 2) & 1)

    @pl.when(wid + (CPW - 1) * NW < NCHUNK)
    def _():
        wait_wb((CPW - 1) & 1)


def _gather(x, rcv, snd):
    ne = rcv.shape[0]
    nchunk = ne // C
    cpw = -(-nchunk // NW)
    f = pl.kernel(
        functools.partial(_gather_body, nchunk, cpw),
        out_type=(jax.ShapeDtypeStruct((ne, D), jnp.float32),
                  jax.ShapeDtypeStruct((ne, D), jnp.float32)),
        mesh=_mesh(),
        scratch_types=[
            pltpu.VMEM((2, C), jnp.int32),
            pltpu.VMEM((2, C), jnp.int32),
            pltpu.VMEM((2, C, D), jnp.float32),
            pltpu.VMEM((2, C, D), jnp.float32),
            pltpu.SemaphoreType.DMA((2,)),
            pltpu.SemaphoreType.DMA((2,)),
            pltpu.SemaphoreType.DMA((2,)),
        ],
    )
    return f(x, rcv, snd)


def _scatter_body(msgA_hbm, msgB_hbm, rcv_hbm, out_hbm,
                  idx_v, rows_v, ztile_v, acc_sh, sems):
    c = lax.axis_index("c")
    s = lax.axis_index("s")
    lo = c * NPSC

    z16 = jnp.zeros((16,), jnp.float32)

    def zrow(i, _):
        for j in range(D // 16):
            ztile_v[i, pl.ds(j * 16, 16)] = z16
        return 0

    lax.fori_loop(0, NPT, zrow, 0)
    pltpu.sync_copy(ztile_v, acc_sh.at[pl.ds(s * NPT, NPT)])

    @pl.when(s == 0)
    def _():
        pltpu.sync_copy(ztile_v.at[pl.ds(0, 8)], acc_sh.at[pl.ds(NPSC, 8)])

    plsc.subcore_barrier()

    per_tile = -(-NCHUNK // NS)

    def fetch(k, buf):
        cid = s + k * NS
        cid2 = jnp.minimum(cid, NCHUNK - 1)
        base = cid2 * C
        half = NCHUNK // 2
        pltpu.async_copy(rcv_hbm.at[pl.ds(base, C)], idx_v.at[buf], sems.at[buf])

        @pl.when(cid2 < half)
        def _():
            pltpu.async_copy(msgA_hbm.at[pl.ds(base, C)], rows_v.at[buf],
                             sems.at[buf])

        @pl.when(cid2 >= half)
        def _():
            pltpu.async_copy(msgB_hbm.at[pl.ds(base - half * C, C)],
                             rows_v.at[buf], sems.at[buf])

    def drain(buf):
        pltpu.make_async_copy(rcv_hbm.at[pl.ds(0, C)], idx_v.at[buf],
                              sems.at[buf]).wait()
        pltpu.make_async_copy(msgA_hbm.at[pl.ds(0, C)], rows_v.at[buf],
                              sems.at[buf]).wait()

    fetch(0, 0)

    def chunk(k, _):
        buf = k & 1
        drain(buf)

        @pl.when(k + 1 < per_tile)
        def _():
            fetch(k + 1, 1 - buf)

        cid = s + k * NS

        @pl.when(cid < NCHUNK)
        def _():
            for j in range(C // 16):
                v = idx_v[buf, pl.ds(j * 16, 16)] - lo
                v = jnp.where((v >= 0) & (v < NPSC), v, NPSC)
                idx_v[buf, pl.ds(j * 16, 16)] = v
            pltpu.sync_copy(rows_v.at[buf], acc_sh.at[idx_v.at[buf]], add=True)
        return 0

    lax.fori_loop(0, per_tile, chunk, 0)
    plsc.subcore_barrier()
    pltpu.sync_copy(acc_sh.at[pl.ds(s * NPT, NPT)],
                    out_hbm.at[pl.ds(lo + s * NPT, NPT)])


def _scatter(msgA, msgB, rcv):
    f = pl.kernel(
        _scatter_body,
        out_type=jax.ShapeDtypeStruct((NPAD, D), jnp.float32),
        mesh=_mesh(),
        scratch_types=[
            pltpu.VMEM((2, C), jnp.int32),
            pltpu.VMEM((2, C, D), jnp.float32),
            pltpu.VMEM((NPT, D), jnp.float32),
            pltpu.VMEM_SHARED((NPSC + 8, D), jnp.float32),
            pltpu.SemaphoreType.DMA((2,)),
        ],
    )
    return f(msgA, msgB, rcv)


def _msg_body(xr, xs, ef, sph, Wr, Ws, Wf, b0, W1, b1, out):
    bf = jnp.bfloat16
    t = (jnp.dot(xr[...].astype(bf), Wr[...], preferred_element_type=jnp.float32)
         + jnp.dot(xs[...].astype(bf), Ws[...], preferred_element_type=jnp.float32)
         + jnp.dot(ef[...], Wf[...], preferred_element_type=jnp.float32))
    y = _silu(_combine(t, sph[...], b0[...]))
    t2 = jnp.dot(y.astype(bf), W1[...], preferred_element_type=jnp.float32)
    out[...] = _silu(_combine(t2, sph[...], b1[...]))


def _msg_mlp(xr2, xs2, ef, sph, Wr, Ws, Wf, b0, W1, b1, offb):
    BE = 5000
    ne = xr2.shape[0]
    grid = (ne // BE,)
    full = lambda r, c_: pl.BlockSpec((r, c_), lambda i: (0, 0))
    row = lambda w: pl.BlockSpec((BE, w), lambda i: (i, 0))
    rowo = lambda w: pl.BlockSpec((BE, w), lambda i: (i + offb, 0))
    return pl.pallas_call(
        _msg_body,
        grid=grid,
        in_specs=[
            row(D), row(D), rowo(D_ADD), rowo(D_ATTR),
            full(D, 4 * D), full(D, 4 * D), full(D_ADD, 4 * D),
            pl.BlockSpec((D,), lambda i: (0,)),
            full(D, 4 * D),
            pl.BlockSpec((D,), lambda i: (0,)),
        ],
        out_specs=row(D),
        out_shape=jax.ShapeDtypeStruct((ne, D), jnp.float32),
    )(xr2, xs2, ef, sph, Wr, Ws, Wf, b0, W1, b1)


def _upd_body(x, agg, attr, Wux, Wua, bu0, Wu1, bu1, out):
    bf = jnp.bfloat16
    t = (jnp.dot(x[...].astype(bf), Wux[...], preferred_element_type=jnp.float32)
         + jnp.dot(agg[...].astype(bf), Wua[...], preferred_element_type=jnp.float32))
    h = _silu(_combine(t, attr[...], bu0[...]))
    t2 = jnp.dot(h.astype(bf), Wu1[...], preferred_element_type=jnp.float32)
    out[...] = x[...] + _combine(t2, attr[...], bu1[...])


def _upd_mlp(x, agg, attr, Wux, Wua, bu0, Wu1, bu1):
    BN = 10000
    grid = (N // BN,)
    full = lambda r, c_: pl.BlockSpec((r, c_), lambda i: (0, 0))
    row = lambda w: pl.BlockSpec((BN, w), lambda i: (i, 0))
    return pl.pallas_call(
        _upd_body,
        grid=grid,
        in_specs=[
            row(D), row(D), row(D_ATTR),
            full(D, 4 * D), full(D, 4 * D),
            pl.BlockSpec((D,), lambda i: (0,)),
            full(D, 4 * D),
            pl.BlockSpec((D,), lambda i: (0,)),
        ],
        out_specs=row(D),
        out_shape=jax.ShapeDtypeStruct((N, D), jnp.float32),
    )(x, agg, attr, Wux, Wua, bu0, Wu1, bu1)


def kernel(x, edge_index, edge_feat, edge_sph, node_attr,
           W_msg0, b_msg0, W_msg1, b_msg1, W_upd0, b_upd0, W_upd1, b_upd1):
    snd = edge_index[0]
    rcv = edge_index[1]

    bf = jnp.bfloat16
    E2 = E // 2
    xrA, xsA = _gather(x, rcv[:E2], snd[:E2])
    xrB, xsB = _gather(x, rcv[E2:], snd[E2:])

    Wr = W_msg0[:D].reshape(D, D_ATTR * D).astype(bf)
    Ws = W_msg0[D:2 * D].reshape(D, D_ATTR * D).astype(bf)
    Wf = W_msg0[2 * D:].reshape(D_ADD, D_ATTR * D).astype(bf)
    W1 = W_msg1.reshape(D, D_ATTR * D).astype(bf)
    ef = edge_feat.astype(bf)
    msgA = _msg_mlp(xrA, xsA, ef, edge_sph, Wr, Ws, Wf, b_msg0, W1, b_msg1, 0)
    nb = E2 // 5000
    msgB = _msg_mlp(xrB, xsB, ef, edge_sph, Wr, Ws, Wf, b_msg0, W1, b_msg1, nb)

    agg = _scatter(msgA, msgB, rcv)

    Wux = W_upd0[:D].reshape(D, D_ATTR * D).astype(bf)
    Wua = W_upd0[D:].reshape(D, D_ATTR * D).astype(bf)
    Wu1 = W_upd1.reshape(D, D_ATTR * D).astype(bf)
    return _upd_mlp(x, agg[:N], node_attr, Wux, Wua, b_upd0, Wu1, b_upd1)


# speedup vs baseline: 1.0067x; 1.0067x over previous
import functools

import jax
import jax.numpy as jnp
import numpy as np
from jax import lax
from jax.experimental import pallas as pl
from jax.experimental.pallas import tpu as pltpu
from jax.experimental.pallas import tpu_sc as plsc

N = 10000
E = 160000
D = 128
D_ADD = 16
D_ATTR = 4

NC, NS = 2, 16
NW = NC * NS
C = 128
NCHUNK = E // C
CPW = -(-NCHUNK // NW)
NPAD = 10240
NPSC = NPAD // NC
NPT = NPSC // NS

_mesh = lambda: plsc.VectorSubcoreMesh(
    core_axis_name="c", subcore_axis_name="s", num_cores=NC, num_subcores=NS)


def _silu(t):
    return t / (1.0 + jnp.exp(-t))


def _combine(t, attr, bias):
    acc = bias
    for b in range(D_ATTR):
        acc = acc + t[:, b * D:(b + 1) * D] * attr[:, b:b + 1]
    return acc


DP = D // 2


def _gather_body(NCHUNK, CPW, x_hbm, rcv_hbm, snd_hbm, xr_hbm, xs_hbm,
                 idxr_v, idxs_v, rowsr_v, rowss_v, isem, gsem, wsem):
    wid = lax.axis_index("s") * NC + lax.axis_index("c")

    def fetch_idx(k, buf):
        cid = wid + jnp.minimum(k, CPW - 1) * NW
        base = jnp.minimum(cid, NCHUNK - 1) * C
        pltpu.async_copy(rcv_hbm.at[pl.ds(base, C)], idxr_v.at[buf], isem.at[buf])
        pltpu.async_copy(snd_hbm.at[pl.ds(base, C)], idxs_v.at[buf], isem.at[buf])

    def wait_idx(buf):
        pltpu.make_async_copy(rcv_hbm.at[pl.ds(0, C)], idxr_v.at[buf],
                              isem.at[buf]).wait()
        pltpu.make_async_copy(snd_hbm.at[pl.ds(0, C)], idxs_v.at[buf],
                              isem.at[buf]).wait()

    def wait_wb(buf):
        pltpu.make_async_copy(rowsr_v.at[buf], xr_hbm.at[pl.ds(0, C)],
                              wsem.at[buf]).wait()
        pltpu.make_async_copy(rowss_v.at[buf], xs_hbm.at[pl.ds(0, C)],
                              wsem.at[buf]).wait()

    fetch_idx(0, 0)

    def chunk(k, _):
        buf = k & 1
        cid = wid + k * NW

        @pl.when(k >= 2)
        def _():
            wait_wb(buf)

        wait_idx(buf)
        cr = pltpu.async_copy(x_hbm.at[idxr_v.at[buf]], rowsr_v.at[buf],
                              gsem.at[buf])
        cs = pltpu.async_copy(x_hbm.at[idxs_v.at[buf]], rowss_v.at[buf],
                              gsem.at[buf])

        @pl.when(k + 1 < CPW)
        def _():
            fetch_idx(k + 1, 1 - buf)

        cr.wait()
        cs.wait()

        @pl.when(cid < NCHUNK)
        def _():
            base = cid * C
            pltpu.async_copy(rowsr_v.at[buf], xr_hbm.at[pl.ds(base, C)],
                             wsem.at[buf])
            pltpu.async_copy(rowss_v.at[buf], xs_hbm.at[pl.ds(base, C)],
                             wsem.at[buf])
        return 0

    lax.fori_loop(0, CPW, chunk, 0)
    wait_wb((CPW - 2) & 1)

    @pl.when(wid + (CPW - 1) * NW < NCHUNK)
    def _():
        wait_wb((CPW - 1) & 1)


def _gather(x, rcv, snd):
    ne = rcv.shape[0]
    nchunk = ne // C
    cpw = -(-nchunk // NW)
    f = pl.kernel(
        functools.partial(_gather_body, nchunk, cpw),
        out_type=(jax.ShapeDtypeStruct((ne, D), jnp.float32),
                  jax.ShapeDtypeStruct((ne, D), jnp.float32)),
        mesh=_mesh(),
        scratch_types=[
            pltpu.VMEM((2, C), jnp.int32),
            pltpu.VMEM((2, C), jnp.int32),
            pltpu.VMEM((2, C, D), jnp.float32),
            pltpu.VMEM((2, C, D), jnp.float32),
            pltpu.SemaphoreType.DMA((2,)),
            pltpu.SemaphoreType.DMA((2,)),
            pltpu.SemaphoreType.DMA((2,)),
        ],
    )
    return f(x, rcv, snd)


def _scatter_body(msgA_hbm, msgB_hbm, rcv_hbm, out_hbm,
                  idx_v, rows_v, ztile_v, acc_sh, sems):
    c = lax.axis_index("c")
    s = lax.axis_index("s")
    lo = c * NPSC

    z16 = jnp.zeros((16,), jnp.float32)

    def zrow(i, _):
        for j in range(D // 16):
            ztile_v[i, pl.ds(j * 16, 16)] = z16
        return 0

    lax.fori_loop(0, NPT, zrow, 0)
    pltpu.sync_copy(ztile_v, acc_sh.at[pl.ds(s * NPT, NPT)])

    @pl.when(s == 0)
    def _():
        pltpu.sync_copy(ztile_v.at[pl.ds(0, 8)], acc_sh.at[pl.ds(NPSC, 8)])

    plsc.subcore_barrier()

    per_tile = -(-NCHUNK // NS)

    def fetch(k, buf):
        cid = s + k * NS
        cid2 = jnp.minimum(cid, NCHUNK - 1)
        base = cid2 * C
        half = NCHUNK // 2
        pltpu.async_copy(rcv_hbm.at[pl.ds(base, C)], idx_v.at[buf], sems.at[buf])

        @pl.when(cid2 < half)
        def _():
            pltpu.async_copy(msgA_hbm.at[pl.ds(base, C)], rows_v.at[buf],
                             sems.at[buf])

        @pl.when(cid2 >= half)
        def _():
            pltpu.async_copy(msgB_hbm.at[pl.ds(base - half * C, C)],
                             rows_v.at[buf], sems.at[buf])

    def drain(buf):
        pltpu.make_async_copy(rcv_hbm.at[pl.ds(0, C)], idx_v.at[buf],
                              sems.at[buf]).wait()
        pltpu.make_async_copy(msgA_hbm.at[pl.ds(0, C)], rows_v.at[buf],
                              sems.at[buf]).wait()

    fetch(0, 0)

    def chunk(k, _):
        buf = k & 1
        drain(buf)

        @pl.when(k + 1 < per_tile)
        def _():
            fetch(k + 1, 1 - buf)

        cid = s + k * NS

        @pl.when(cid < NCHUNK)
        def _():
            for j in range(C // 16):
                v = idx_v[buf, pl.ds(j * 16, 16)] - lo
                v = jnp.where((v >= 0) & (v < NPSC), v, NPSC)
                idx_v[buf, pl.ds(j * 16, 16)] = v
            pltpu.sync_copy(rows_v.at[buf], acc_sh.at[idx_v.at[buf]], add=True)
        return 0

    lax.fori_loop(0, per_tile, chunk, 0)
    plsc.subcore_barrier()
    pltpu.sync_copy(acc_sh.at[pl.ds(s * NPT, NPT)],
                    out_hbm.at[pl.ds(lo + s * NPT, NPT)])


def _scatter(msgA, msgB, rcv):
    f = pl.kernel(
        _scatter_body,
        out_type=jax.ShapeDtypeStruct((NPAD, D), jnp.float32),
        mesh=_mesh(),
        scratch_types=[
            pltpu.VMEM((2, C), jnp.int32),
            pltpu.VMEM((2, C, D), jnp.float32),
            pltpu.VMEM((NPT, D), jnp.float32),
            pltpu.VMEM_SHARED((NPSC + 8, D), jnp.float32),
            pltpu.SemaphoreType.DMA((2,)),
        ],
    )
    return f(msgA, msgB, rcv)


def _msg_body(xr, xs, ef, sph, Wr, Ws, Wf, b0, W1, b1, out):
    bf = jnp.bfloat16
    t = (jnp.dot(xr[...].astype(bf), Wr[...], preferred_element_type=jnp.float32)
         + jnp.dot(xs[...].astype(bf), Ws[...], preferred_element_type=jnp.float32)
         + jnp.dot(ef[...], Wf[...], preferred_element_type=jnp.float32))
    y = _silu(_combine(t, sph[...], b0[...]))
    t2 = jnp.dot(y.astype(bf), W1[...], preferred_element_type=jnp.float32)
    out[...] = _silu(_combine(t2, sph[...], b1[...]))


def _msg_mlp(xr2, xs2, ef, sph, Wr, Ws, Wf, b0, W1, b1, offb):
    BE = 4000
    ne = xr2.shape[0]
    grid = (ne // BE,)
    full = lambda r, c_: pl.BlockSpec((r, c_), lambda i: (0, 0))
    row = lambda w: pl.BlockSpec((BE, w), lambda i: (i, 0))
    rowo = lambda w: pl.BlockSpec((BE, w), lambda i: (i + offb, 0))
    return pl.pallas_call(
        _msg_body,
        grid=grid,
        in_specs=[
            row(D), row(D), rowo(D_ADD), rowo(D_ATTR),
            full(D, 4 * D), full(D, 4 * D), full(D_ADD, 4 * D),
            pl.BlockSpec((D,), lambda i: (0,)),
            full(D, 4 * D),
            pl.BlockSpec((D,), lambda i: (0,)),
        ],
        out_specs=row(D),
        out_shape=jax.ShapeDtypeStruct((ne, D), jnp.float32),
    )(xr2, xs2, ef, sph, Wr, Ws, Wf, b0, W1, b1)


def _upd_body(x, agg, attr, Wux, Wua, bu0, Wu1, bu1, out):
    bf = jnp.bfloat16
    t = (jnp.dot(x[...].astype(bf), Wux[...], preferred_element_type=jnp.float32)
         + jnp.dot(agg[...].astype(bf), Wua[...], preferred_element_type=jnp.float32))
    h = _silu(_combine(t, attr[...], bu0[...]))
    t2 = jnp.dot(h.astype(bf), Wu1[...], preferred_element_type=jnp.float32)
    out[...] = x[...] + _combine(t2, attr[...], bu1[...])


def _upd_mlp(x, agg, attr, Wux, Wua, bu0, Wu1, bu1):
    BN = 5000
    grid = (N // BN,)
    full = lambda r, c_: pl.BlockSpec((r, c_), lambda i: (0, 0))
    row = lambda w: pl.BlockSpec((BN, w), lambda i: (i, 0))
    return pl.pallas_call(
        _upd_body,
        grid=grid,
        in_specs=[
            row(D), row(D), row(D_ATTR),
            full(D, 4 * D), full(D, 4 * D),
            pl.BlockSpec((D,), lambda i: (0,)),
            full(D, 4 * D),
            pl.BlockSpec((D,), lambda i: (0,)),
        ],
        out_specs=row(D),
        out_shape=jax.ShapeDtypeStruct((N, D), jnp.float32),
    )(x, agg, attr, Wux, Wua, bu0, Wu1, bu1)


def kernel(x, edge_index, edge_feat, edge_sph, node_attr,
           W_msg0, b_msg0, W_msg1, b_msg1, W_upd0, b_upd0, W_upd1, b_upd1):
    snd = edge_index[0]
    rcv = edge_index[1]

    bf = jnp.bfloat16
    E2 = E // 2
    xrA, xsA = _gather(x, rcv[:E2], snd[:E2])
    xrB, xsB = _gather(x, rcv[E2:], snd[E2:])

    Wr = W_msg0[:D].reshape(D, D_ATTR * D).astype(bf)
    Ws = W_msg0[D:2 * D].reshape(D, D_ATTR * D).astype(bf)
    Wf = W_msg0[2 * D:].reshape(D_ADD, D_ATTR * D).astype(bf)
    W1 = W_msg1.reshape(D, D_ATTR * D).astype(bf)
    ef = edge_feat.astype(bf)
    msgA = _msg_mlp(xrA, xsA, ef, edge_sph, Wr, Ws, Wf, b_msg0, W1, b_msg1, 0)
    nb = E2 // 4000
    msgB = _msg_mlp(xrB, xsB, ef, edge_sph, Wr, Ws, Wf, b_msg0, W1, b_msg1, nb)

    agg = _scatter(msgA, msgB, rcv)

    Wux = W_upd0[:D].reshape(D, D_ATTR * D).astype(bf)
    Wua = W_upd0[D:].reshape(D, D_ATTR * D).astype(bf)
    Wu1 = W_upd1.reshape(D, D_ATTR * D).astype(bf)
    return _upd_mlp(x, agg[:N], node_attr, Wux, Wua, b_upd0, Wu1, b_upd1)
